# baseline (speedup 1.0000x reference)
import jax
import jax.numpy as jnp
from jax import lax
from jax.experimental import pallas as pl
from jax.experimental.pallas import tpu as pltpu
from jax._src.pallas.mosaic import sc_core as plsc
from jax._src.pallas.mosaic import sc_primitives as plsc_p

_ROWS = 3 * 1080
_COLS = 1920
_SC_ROWS = 648
_TC_ROWS = _ROWS - _SC_ROWS
_TC_BM = 1032
_N_TILES = 32
_TILE_ELEMS = _SC_ROWS * _COLS // _N_TILES


def _tc_body(idx_ref, a_ref, b_ref, x_ref, o_ref):
    i = idx_ref[0]
    scale = jnp.exp(a_ref[i])
    shift = b_ref[i]
    for r in range(0, _TC_BM, 344):
        o_ref[pl.ds(r, 344), :] = x_ref[pl.ds(r, 344), :] * scale + shift


def _sc_body(idx_ref, a_ref, b_ref, x_ref, o_ref,
             idxb, ab, bb, ibuf, obuf, s0, s1, s2, s3, s4):
    c = lax.axis_index("c")
    s = lax.axis_index("s")
    tile = c * 16 + s
    obase = tile * _TILE_ELEMS
    xbase = _TC_ROWS * _COLS + obase
    cp_x = pltpu.make_async_copy(x_ref.at[pl.ds(xbase, _TILE_ELEMS)], ibuf, s0)
    cp_i = pltpu.make_async_copy(idx_ref, idxb, s1)
    cp_a = pltpu.make_async_copy(a_ref, ab, s2)
    cp_b = pltpu.make_async_copy(b_ref, bb, s3)
    cp_x.start()
    cp_i.start()
    cp_a.start()
    cp_b.start()
    cp_i.wait()
    cp_a.wait()
    cp_b.wait()
    iv = idxb[...]
    scale = jnp.exp(plsc_p.load_gather(ab, [iv]))
    shift = plsc_p.load_gather(bb, [iv])
    cp_x.wait()

    def step(i, carry):
        off = i * 16
        obuf[pl.ds(off, 16)] = ibuf[pl.ds(off, 16)] * scale + shift
        return carry

    lax.fori_loop(0, _TILE_ELEMS // 16, step, 0)
    cp_o = pltpu.make_async_copy(obuf, o_ref.at[pl.ds(obase, _TILE_ELEMS)], s4)
    cp_o.start()
    cp_o.wait()


def kernel(rendered_image, cur_index, exposure_a, exposure_b):
    x2d = rendered_image.reshape(_ROWS, _COLS)
    a_flat = exposure_a.reshape(-1)
    b_flat = exposure_b.reshape(-1)
    x_flat = x2d.reshape(-1)

    out_tc = pl.pallas_call(
        _tc_body,
        grid=(pl.cdiv(_TC_ROWS, _TC_BM),),
        in_specs=[
            pl.BlockSpec(memory_space=pltpu.SMEM),
            pl.BlockSpec(memory_space=pltpu.SMEM),
            pl.BlockSpec(memory_space=pltpu.SMEM),
            pl.BlockSpec((_TC_BM, _COLS), lambda i: (i, 0)),
        ],
        out_specs=pl.BlockSpec((_TC_BM, _COLS), lambda i: (i, 0)),
        out_shape=jax.ShapeDtypeStruct((_TC_ROWS, _COLS), jnp.float32),
    )(cur_index, a_flat, b_flat, x2d)

    sc_call = pl.kernel(
        _sc_body,
        out_type=jax.ShapeDtypeStruct((_SC_ROWS * _COLS,), jnp.float32),
        mesh=plsc.VectorSubcoreMesh(core_axis_name="c", subcore_axis_name="s"),
        scratch_types=[
            pltpu.VMEM((16,), jnp.int32),
            pltpu.VMEM((1000,), jnp.float32),
            pltpu.VMEM((1000,), jnp.float32),
            pltpu.VMEM((_TILE_ELEMS,), jnp.float32),
            pltpu.VMEM((_TILE_ELEMS,), jnp.float32),
            pltpu.SemaphoreType.DMA,
            pltpu.SemaphoreType.DMA,
            pltpu.SemaphoreType.DMA,
            pltpu.SemaphoreType.DMA,
            pltpu.SemaphoreType.DMA,
        ],
        compiler_params=pltpu.CompilerParams(needs_layout_passes=False),
    )
    idx16 = jnp.broadcast_to(cur_index, (16,))
    out_sc = sc_call(idx16, a_flat, b_flat, x_flat)

    out = jnp.concatenate([out_tc, out_sc.reshape(_SC_ROWS, _COLS)], axis=0)
    return out.reshape(rendered_image.shape)


# speedup vs baseline: 5.3118x; 5.3118x over previous
import jax
import jax.numpy as jnp
from jax.experimental import pallas as pl
from jax.experimental.pallas import tpu as pltpu

_ROWS = 3 * 1080
_COLS = 1920
_BM = 1224
_SUB = 408


def _body(idx_ref, a_ref, b_ref, x_ref, o_ref):
    i = idx_ref[0]
    scale = jnp.exp(a_ref[i])
    shift = b_ref[i]
    for r in range(0, _BM, _SUB):
        o_ref[pl.ds(r, _SUB), :] = x_ref[pl.ds(r, _SUB), :] * scale + shift


def kernel(rendered_image, cur_index, exposure_a, exposure_b):
    x2d = rendered_image.reshape(_ROWS, _COLS)
    out = pl.pallas_call(
        _body,
        grid=(pl.cdiv(_ROWS, _BM),),
        in_specs=[
            pl.BlockSpec(memory_space=pltpu.SMEM),
            pl.BlockSpec(memory_space=pltpu.SMEM),
            pl.BlockSpec(memory_space=pltpu.SMEM),
            pl.BlockSpec((_BM, _COLS), lambda i: (i, 0)),
        ],
        out_specs=pl.BlockSpec((_BM, _COLS), lambda i: (i, 0)),
        out_shape=jax.ShapeDtypeStruct((_ROWS, _COLS), jnp.float32),
        compiler_params=pltpu.CompilerParams(vmem_limit_bytes=100 * 1024 * 1024),
    )(cur_index, exposure_a.reshape(-1), exposure_b.reshape(-1), x2d)
    return out.reshape(rendered_image.shape)


# speedup vs baseline: 5.3349x; 1.0044x over previous
import jax
import jax.numpy as jnp
from jax.experimental import pallas as pl
from jax.experimental.pallas import tpu as pltpu

_ROWS = 3 * 1080
_COLS = 1920
_BM = 1312
_SUB = 328


def _body(idx_ref, a_ref, b_ref, x_ref, o_ref):
    i = idx_ref[0]
    scale = jnp.exp(a_ref[i])
    shift = b_ref[i]
    for r in range(0, _BM, _SUB):
        o_ref[pl.ds(r, _SUB), :] = x_ref[pl.ds(r, _SUB), :] * scale + shift


def kernel(rendered_image, cur_index, exposure_a, exposure_b):
    x2d = rendered_image.reshape(_ROWS, _COLS)
    out = pl.pallas_call(
        _body,
        grid=(pl.cdiv(_ROWS, _BM),),
        in_specs=[
            pl.BlockSpec(memory_space=pltpu.SMEM),
            pl.BlockSpec(memory_space=pltpu.SMEM),
            pl.BlockSpec(memory_space=pltpu.SMEM),
            pl.BlockSpec((_BM, _COLS), lambda i: (i, 0)),
        ],
        out_specs=pl.BlockSpec((_BM, _COLS), lambda i: (i, 0)),
        out_shape=jax.ShapeDtypeStruct((_ROWS, _COLS), jnp.float32),
        compiler_params=pltpu.CompilerParams(vmem_limit_bytes=100 * 1024 * 1024),
    )(cur_index, exposure_a.reshape(-1), exposure_b.reshape(-1), x2d)
    return out.reshape(rendered_image.shape)
